# baseline (speedup 1.0000x reference)
import functools

import jax
import jax.numpy as jnp
from jax import lax
from jax.experimental import pallas as pl
from jax.experimental.pallas import tpu as pltpu
from jax.experimental.pallas import tpu_sc as plsc

_NB = 16384
_L = 50
_D = 64
_NC = 2
_NS = 16
_NW = _NC * _NS
_RPW = _NB // _NW
_CR = 8
_N = _RPW // _CR
_NBUF = 2

_mesh = plsc.VectorSubcoreMesh(core_axis_name="c", subcore_axis_name="s")


@functools.partial(
    pl.kernel,
    mesh=_mesh,
    out_type=jax.ShapeDtypeStruct((_NB, _L, _D), jnp.float32),
    scratch_types=(
        [pltpu.VMEM((_CR, _L), jnp.int32) for _ in range(_NBUF)]
        + [pltpu.VMEM((_CR, _L, 2 * _D), jnp.float32) for _ in range(_NBUF)]
        + [pltpu.SemaphoreType.DMA for _ in range(2 * _NBUF)]
    ),
    compiler_params=pltpu.CompilerParams(use_tc_tiling_on_sc=False),
)
def _embed_sc(idx_hbm, table_hbm, out_hbm, *scratch):
    idxb = scratch[0:_NBUF]
    rows = scratch[_NBUF:2 * _NBUF]
    gsem = scratch[2 * _NBUF:3 * _NBUF]
    ssem = scratch[3 * _NBUF:4 * _NBUF]

    wid = lax.axis_index("s") * _NC + lax.axis_index("c")
    base = wid * _RPW

    def load_gather(i, b):
        off = base + i * _CR
        pltpu.sync_copy(idx_hbm.at[pl.ds(off, _CR)], idxb[b])
        for j in range(_CR):
            pltpu.async_copy(table_hbm.at[idxb[b].at[j]], rows[b].at[j],
                             gsem[b])

    def wait_gather(b):
        for j in range(_CR):
            pltpu.make_async_copy(table_hbm.at[idxb[b].at[j]], rows[b].at[j],
                                  gsem[b]).wait()

    def start_store(i, b):
        off = base + i * _CR
        pltpu.async_copy(rows[b].at[:, :, pl.ds(0, _D)],
                         out_hbm.at[pl.ds(off, _CR)], ssem[b])

    def wait_store(i, b):
        off = base + i * _CR
        pltpu.make_async_copy(rows[b].at[:, :, pl.ds(0, _D)],
                              out_hbm.at[pl.ds(off, _CR)], ssem[b]).wait()

    for b in range(_NBUF):
        load_gather(b, b)

    def body(g, carry):
        i0 = g * _NBUF
        for b in range(_NBUF):
            wait_gather(b)
            start_store(i0 + b, b)
        for b in range(_NBUF):
            wait_store(i0 + b, b)
            load_gather(i0 + b + _NBUF, b)
        return carry

    lax.fori_loop(0, _N // _NBUF - 1, body, 0)

    i0 = _N - _NBUF
    for b in range(_NBUF):
        wait_gather(b)
        start_store(i0 + b, b)
    for b in range(_NBUF):
        wait_store(i0 + b, b)


def kernel(x, weight):
    if x.dtype != jnp.int32:
        x = x.astype(jnp.int32)
    wp = jnp.pad(weight, ((0, 0), (0, _D)))
    return _embed_sc(x, wp)


# speedup vs baseline: 1.0735x; 1.0735x over previous
import functools

import jax
import jax.numpy as jnp
from jax import lax
from jax.experimental import pallas as pl
from jax.experimental.pallas import tpu as pltpu
from jax.experimental.pallas import tpu_sc as plsc

_NB = 16384
_L = 50
_D = 64
_NC = 2
_NS = 16
_NW = _NC * _NS
_RPW = _NB // _NW
_CR = 16
_N = _RPW // _CR
_NBUF = 2

_mesh = plsc.VectorSubcoreMesh(core_axis_name="c", subcore_axis_name="s")


@functools.partial(
    pl.kernel,
    mesh=_mesh,
    out_type=jax.ShapeDtypeStruct((_NB, _L, _D), jnp.float32),
    scratch_types=(
        [pltpu.VMEM((_CR, _L), jnp.int32) for _ in range(_NBUF)]
        + [pltpu.VMEM((_CR, _L, _D), jnp.float32) for _ in range(_NBUF)]
        + [pltpu.SemaphoreType.DMA for _ in range(2 * _NBUF)]
    ),
    compiler_params=pltpu.CompilerParams(use_tc_tiling_on_sc=False),
)
def _embed_sc(idx_hbm, table_hbm, out_hbm, *scratch):
    idxb = scratch[0:_NBUF]
    rows = scratch[_NBUF:2 * _NBUF]
    gsem = scratch[2 * _NBUF:3 * _NBUF]
    ssem = scratch[3 * _NBUF:4 * _NBUF]

    wid = lax.axis_index("s") * _NC + lax.axis_index("c")
    base = wid * _RPW

    def load_gather(i, b):
        off = base + i * _CR
        pltpu.sync_copy(idx_hbm.at[pl.ds(off, _CR)], idxb[b])
        for j in range(_CR):
            pltpu.async_copy(table_hbm.at[idxb[b].at[j]], rows[b].at[j],
                             gsem[b])

    def wait_gather(b):
        for j in range(_CR):
            pltpu.make_async_copy(table_hbm.at[idxb[b].at[j]], rows[b].at[j],
                                  gsem[b]).wait()

    def start_store(i, b):
        off = base + i * _CR
        pltpu.async_copy(rows[b], out_hbm.at[pl.ds(off, _CR)], ssem[b])

    def wait_store(i, b):
        off = base + i * _CR
        pltpu.make_async_copy(rows[b], out_hbm.at[pl.ds(off, _CR)],
                              ssem[b]).wait()

    for b in range(_NBUF):
        load_gather(b, b)

    def body(g, carry):
        i0 = g * _NBUF
        for b in range(_NBUF):
            wait_gather(b)
            start_store(i0 + b, b)
        for b in range(_NBUF):
            wait_store(i0 + b, b)
            load_gather(i0 + b + _NBUF, b)
        return carry

    lax.fori_loop(0, _N // _NBUF - 1, body, 0)

    i0 = _N - _NBUF
    for b in range(_NBUF):
        wait_gather(b)
        start_store(i0 + b, b)
    for b in range(_NBUF):
        wait_store(i0 + b, b)


def kernel(x, weight):
    if x.dtype != jnp.int32:
        x = x.astype(jnp.int32)
    return _embed_sc(x, weight)


# speedup vs baseline: 1.0879x; 1.0134x over previous
import functools

import jax
import jax.numpy as jnp
from jax import lax
from jax.experimental import pallas as pl
from jax.experimental.pallas import tpu as pltpu
from jax.experimental.pallas import tpu_sc as plsc

_NBATCH = 16384
_L = 50
_D = 64
_NC = 2
_NS = 16
_NW = _NC * _NS
_BB = 128
_NBG = _NBATCH // _BB
_NBLK = _L * _NBG
_BLKW = _NBLK // _NW
_NBUF = 2

_mesh = plsc.VectorSubcoreMesh(core_axis_name="c", subcore_axis_name="s")


@functools.partial(
    pl.kernel,
    mesh=_mesh,
    out_type=jax.ShapeDtypeStruct((_L, _NBG, _BB, _D), jnp.float32),
    scratch_types=(
        [pltpu.VMEM((1, _BB), jnp.int32) for _ in range(_NBUF)]
        + [pltpu.VMEM((1, 1, _BB, _D), jnp.float32) for _ in range(_NBUF)]
        + [pltpu.SemaphoreType.DMA for _ in range(2 * _NBUF)]
    ),
    compiler_params=pltpu.CompilerParams(use_tc_tiling_on_sc=False),
)
def _embed_sc(idx_hbm, table_hbm, out_hbm, *scratch):
    idxb = scratch[0:_NBUF]
    rows = scratch[_NBUF:2 * _NBUF]
    gsem = scratch[2 * _NBUF:3 * _NBUF]
    ssem = scratch[3 * _NBUF:4 * _NBUF]

    wid = lax.axis_index("s") * _NC + lax.axis_index("c")
    base = wid * _BLKW

    def coords(i):
        bid = base + i
        return bid // _NBG, bid % _NBG

    def load_gather(i, b):
        l, bg = coords(i)
        pltpu.sync_copy(idx_hbm.at[pl.ds(l, 1), pl.ds(bg * _BB, _BB)],
                        idxb[b])
        pltpu.async_copy(table_hbm.at[idxb[b].at[0]], rows[b].at[0, 0],
                         gsem[b])

    def wait_gather(b):
        pltpu.make_async_copy(table_hbm.at[idxb[b].at[0]], rows[b].at[0, 0],
                              gsem[b]).wait()

    def start_store(i, b):
        l, bg = coords(i)
        pltpu.async_copy(rows[b], out_hbm.at[pl.ds(l, 1), pl.ds(bg, 1)],
                         ssem[b])

    def wait_store(i, b):
        l, bg = coords(i)
        pltpu.make_async_copy(rows[b], out_hbm.at[pl.ds(l, 1), pl.ds(bg, 1)],
                              ssem[b]).wait()

    for b in range(_NBUF):
        load_gather(b, b)

    def body(g, carry):
        for b in range(_NBUF):
            i = g * _NBUF + b
            wait_gather(b)

            @pl.when(i >= _NBUF)
            def _():
                wait_store(i - _NBUF, b)

            start_store(i, b)

            @pl.when(i + _NBUF < _BLKW)
            def _():
                load_gather(i + _NBUF, b)

        return carry

    lax.fori_loop(0, _BLKW // _NBUF, body, 0)

    for b in range(_NBUF):
        wait_store(_BLKW - _NBUF + b, b)


def kernel(x, weight):
    if x.dtype != jnp.int32:
        x = x.astype(jnp.int32)
    xt = jnp.transpose(x, (1, 0))
    out4 = _embed_sc(xt, weight)
    return jnp.transpose(out4.reshape(_L, _NBATCH, _D), (1, 0, 2))


# speedup vs baseline: 1.1254x; 1.0345x over previous
import functools

import jax
import jax.numpy as jnp
from jax import lax
from jax.experimental import pallas as pl
from jax.experimental.pallas import tpu as pltpu
from jax.experimental.pallas import tpu_sc as plsc

_NBATCH = 16384
_L = 50
_D = 64
_NC = 2
_NS = 16
_NW = _NC * _NS
_BB = 512
_NBG = _NBATCH // _BB
_NBLK = _L * _NBG
_BLKW = _NBLK // _NW
_NBUF = 2

_mesh = plsc.VectorSubcoreMesh(core_axis_name="c", subcore_axis_name="s")


@functools.partial(
    pl.kernel,
    mesh=_mesh,
    out_type=jax.ShapeDtypeStruct((_L, _NBG, _BB, _D), jnp.float32),
    scratch_types=(
        [pltpu.VMEM((1, _BB), jnp.int32) for _ in range(_NBUF)]
        + [pltpu.VMEM((1, 1, _BB, _D), jnp.float32) for _ in range(_NBUF)]
        + [pltpu.SemaphoreType.DMA for _ in range(2 * _NBUF)]
    ),
    compiler_params=pltpu.CompilerParams(use_tc_tiling_on_sc=False),
)
def _embed_sc(idx_hbm, table_hbm, out_hbm, *scratch):
    idxb = scratch[0:_NBUF]
    rows = scratch[_NBUF:2 * _NBUF]
    gsem = scratch[2 * _NBUF:3 * _NBUF]
    ssem = scratch[3 * _NBUF:4 * _NBUF]

    wid = lax.axis_index("s") * _NC + lax.axis_index("c")
    base = wid * _BLKW

    def coords(i):
        bid = base + i
        return bid // _NBG, bid % _NBG

    def load_gather(i, b):
        l, bg = coords(i)
        pltpu.sync_copy(idx_hbm.at[pl.ds(l, 1), pl.ds(bg * _BB, _BB)],
                        idxb[b])
        pltpu.async_copy(table_hbm.at[idxb[b].at[0]], rows[b].at[0, 0],
                         gsem[b])

    def wait_gather(b):
        pltpu.make_async_copy(table_hbm.at[idxb[b].at[0]], rows[b].at[0, 0],
                              gsem[b]).wait()

    def start_store(i, b):
        l, bg = coords(i)
        pltpu.async_copy(rows[b], out_hbm.at[pl.ds(l, 1), pl.ds(bg, 1)],
                         ssem[b])

    def wait_store(i, b):
        l, bg = coords(i)
        pltpu.make_async_copy(rows[b], out_hbm.at[pl.ds(l, 1), pl.ds(bg, 1)],
                              ssem[b]).wait()

    for b in range(_NBUF):
        load_gather(b, b)

    def body(g, carry):
        for b in range(_NBUF):
            i = g * _NBUF + b
            wait_gather(b)

            @pl.when(i >= _NBUF)
            def _():
                wait_store(i - _NBUF, b)

            start_store(i, b)

            @pl.when(i + _NBUF < _BLKW)
            def _():
                load_gather(i + _NBUF, b)

        return carry

    lax.fori_loop(0, _BLKW // _NBUF, body, 0)

    for b in range(_NBUF):
        wait_store(_BLKW - _NBUF + b, b)


def kernel(x, weight):
    if x.dtype != jnp.int32:
        x = x.astype(jnp.int32)
    xt = jnp.transpose(x, (1, 0))
    out4 = _embed_sc(xt, weight)
    return jnp.transpose(out4.reshape(_L, _NBATCH, _D), (1, 0, 2))


# speedup vs baseline: 1.1269x; 1.0013x over previous
import functools

import jax
import jax.numpy as jnp
from jax import lax
from jax.experimental import pallas as pl
from jax.experimental.pallas import tpu as pltpu
from jax.experimental.pallas import tpu_sc as plsc

_NBATCH = 16384
_L = 50
_D = 64
_NC = 2
_NS = 16
_NW = _NC * _NS
_BB = 256
_NBG = _NBATCH // _BB
_NBLK = _L * _NBG
_BLKW = _NBLK // _NW
_NBUF = 4

_mesh = plsc.VectorSubcoreMesh(core_axis_name="c", subcore_axis_name="s")


@functools.partial(
    pl.kernel,
    mesh=_mesh,
    out_type=jax.ShapeDtypeStruct((_L, _NBG, _BB, _D), jnp.float32),
    scratch_types=(
        [pltpu.VMEM((1, _BB), jnp.int32) for _ in range(_NBUF)]
        + [pltpu.VMEM((1, 1, _BB, _D), jnp.float32) for _ in range(_NBUF)]
        + [pltpu.SemaphoreType.DMA for _ in range(2 * _NBUF)]
    ),
    compiler_params=pltpu.CompilerParams(use_tc_tiling_on_sc=False),
)
def _embed_sc(idx_hbm, table_hbm, out_hbm, *scratch):
    idxb = scratch[0:_NBUF]
    rows = scratch[_NBUF:2 * _NBUF]
    gsem = scratch[2 * _NBUF:3 * _NBUF]
    ssem = scratch[3 * _NBUF:4 * _NBUF]

    wid = lax.axis_index("s") * _NC + lax.axis_index("c")
    base = wid * _BLKW

    def coords(i):
        bid = base + i
        return bid // _NBG, bid % _NBG

    def load_gather(i, b):
        l, bg = coords(i)
        pltpu.sync_copy(idx_hbm.at[pl.ds(l, 1), pl.ds(bg * _BB, _BB)],
                        idxb[b])
        pltpu.async_copy(table_hbm.at[idxb[b].at[0]], rows[b].at[0, 0],
                         gsem[b])

    def wait_gather(b):
        pltpu.make_async_copy(table_hbm.at[idxb[b].at[0]], rows[b].at[0, 0],
                              gsem[b]).wait()

    def start_store(i, b):
        l, bg = coords(i)
        pltpu.async_copy(rows[b], out_hbm.at[pl.ds(l, 1), pl.ds(bg, 1)],
                         ssem[b])

    def wait_store(i, b):
        l, bg = coords(i)
        pltpu.make_async_copy(rows[b], out_hbm.at[pl.ds(l, 1), pl.ds(bg, 1)],
                              ssem[b]).wait()

    for b in range(_NBUF):
        load_gather(b, b)

    def body(g, carry):
        for b in range(_NBUF):
            i = g * _NBUF + b
            wait_gather(b)

            @pl.when(i >= _NBUF)
            def _():
                wait_store(i - _NBUF, b)

            start_store(i, b)

            @pl.when(i + _NBUF < _BLKW)
            def _():
                load_gather(i + _NBUF, b)

        return carry

    lax.fori_loop(0, _BLKW // _NBUF, body, 0)

    for b in range(_NBUF):
        wait_store(_BLKW - _NBUF + b, b)


def kernel(x, weight):
    if x.dtype != jnp.int32:
        x = x.astype(jnp.int32)
    xt = jnp.transpose(x, (1, 0))
    out4 = _embed_sc(xt, weight)
    return jnp.transpose(out4.reshape(_L, _NBATCH, _D), (1, 0, 2))
